# baseline (speedup 1.0000x reference)
import functools

import jax
import jax.numpy as jnp
from jax.experimental import pallas as pl
from jax.experimental.pallas import tpu as pltpu

_BB = 8


def _body(q_ref, d_ref, qid_ref, qw_ref, didt_ref, dwt_ref, qcls_ref,
          dcls_ref, out_ref, *, kd_count):
    cls = jnp.sum(qcls_ref[...] * dcls_ref[...], axis=1)
    lane = jax.lax.broadcasted_iota(jnp.int32, (1, 1, _BB), 2)
    vals = jnp.zeros((1, 1, _BB), jnp.float32)
    for t in range(_BB):
        q = q_ref[t]
        dm = d_ref[t]
        s = jax.lax.dot_general(
            q, dm, (((1,), (1,)), ((), ())),
            preferred_element_type=jnp.float32,
            precision=jax.lax.Precision.HIGHEST)
        qid = qid_ref[t]
        m = None
        for kd in range(kd_count):
            didk = didt_ref[t, kd, :]
            dwk = dwt_ref[t, kd, :]
            match = qid[:, None] == didk[None, :]
            val = jnp.where(match, s * dwk[None, :], 0.0)
            m = val if m is None else jnp.maximum(m, val)
        mi = jnp.max(m, axis=1)
        tok = jnp.sum(qw_ref[t] * mi)
        vals = jnp.where(lane == t, tok, vals)
    out_ref[...] = vals + cls.reshape(1, 1, _BB)


def kernel(query_expert_repr, query_expert_weights, query_expert_ids,
           doc_expert_repr, doc_expert_weights, doc_expert_ids,
           query_cls_repr, doc_cls_repr):
    B, Lq, D = query_expert_repr.shape
    _, Ld, Kd = doc_expert_ids.shape
    G = B // _BB

    qid = query_expert_ids.reshape(B, Lq).astype(jnp.int32)
    qw = query_expert_weights.reshape(B, Lq)
    didt = jnp.swapaxes(doc_expert_ids, 1, 2)
    dwt = jnp.swapaxes(doc_expert_weights, 1, 2)

    out = pl.pallas_call(
        functools.partial(_body, kd_count=Kd),
        grid=(G,),
        in_specs=[
            pl.BlockSpec((_BB, Lq, D), lambda g: (g, 0, 0)),
            pl.BlockSpec((_BB, Ld, D), lambda g: (g, 0, 0)),
            pl.BlockSpec((_BB, Lq), lambda g: (g, 0)),
            pl.BlockSpec((_BB, Lq), lambda g: (g, 0)),
            pl.BlockSpec((_BB, Kd, Ld), lambda g: (g, 0, 0)),
            pl.BlockSpec((_BB, Kd, Ld), lambda g: (g, 0, 0)),
            pl.BlockSpec((_BB, D), lambda g: (g, 0)),
            pl.BlockSpec((_BB, D), lambda g: (g, 0)),
        ],
        out_specs=pl.BlockSpec((1, 1, _BB), lambda g: (g, 0, 0)),
        out_shape=jax.ShapeDtypeStruct((G, 1, _BB), jnp.float32),
    )(query_expert_repr, doc_expert_repr, qid, qw, didt, dwt,
      query_cls_repr, doc_cls_repr)
    return out.reshape(B)


# speedup vs baseline: 1.3054x; 1.3054x over previous
import functools

import jax
import jax.numpy as jnp
from jax.experimental import pallas as pl
from jax.experimental.pallas import tpu as pltpu

_BB = 8


def _body(q_ref, d_ref, qid_ref, qw_ref, didt_ref, dwt_ref, qcls_ref,
          dcls_ref, out_ref, *, kd_count):
    cls = jnp.sum(qcls_ref[...] * dcls_ref[...], axis=1)
    lane = jax.lax.broadcasted_iota(jnp.int32, (1, 1, _BB), 2)
    vals = jnp.zeros((1, 1, _BB), jnp.float32)
    for t in range(_BB):
        q = q_ref[t]
        dm = d_ref[t]
        s = jax.lax.dot_general(
            q, dm, (((1,), (1,)), ((), ())),
            preferred_element_type=jnp.float32)
        qid = qid_ref[t]
        m = None
        for kd in range(kd_count):
            didk = didt_ref[t, kd, :]
            dwk = dwt_ref[t, kd, :]
            match = qid[:, None] == didk[None, :]
            val = jnp.where(match, s * dwk[None, :], 0.0)
            m = val if m is None else jnp.maximum(m, val)
        mi = jnp.max(m, axis=1)
        tok = jnp.sum(qw_ref[t] * mi)
        vals = jnp.where(lane == t, tok, vals)
    out_ref[...] = vals + cls.reshape(1, 1, _BB)


def kernel(query_expert_repr, query_expert_weights, query_expert_ids,
           doc_expert_repr, doc_expert_weights, doc_expert_ids,
           query_cls_repr, doc_cls_repr):
    B, Lq, D = query_expert_repr.shape
    _, Ld, Kd = doc_expert_ids.shape
    G = B // _BB

    qid = query_expert_ids.reshape(B, Lq).astype(jnp.int32)
    qw = query_expert_weights.reshape(B, Lq)
    didt = jnp.swapaxes(doc_expert_ids, 1, 2)
    dwt = jnp.swapaxes(doc_expert_weights, 1, 2)

    out = pl.pallas_call(
        functools.partial(_body, kd_count=Kd),
        grid=(G,),
        in_specs=[
            pl.BlockSpec((_BB, Lq, D), lambda g: (g, 0, 0)),
            pl.BlockSpec((_BB, Ld, D), lambda g: (g, 0, 0)),
            pl.BlockSpec((_BB, Lq), lambda g: (g, 0)),
            pl.BlockSpec((_BB, Lq), lambda g: (g, 0)),
            pl.BlockSpec((_BB, Kd, Ld), lambda g: (g, 0, 0)),
            pl.BlockSpec((_BB, Kd, Ld), lambda g: (g, 0, 0)),
            pl.BlockSpec((_BB, D), lambda g: (g, 0)),
            pl.BlockSpec((_BB, D), lambda g: (g, 0)),
        ],
        out_specs=pl.BlockSpec((1, 1, _BB), lambda g: (g, 0, 0)),
        out_shape=jax.ShapeDtypeStruct((G, 1, _BB), jnp.float32),
    )(query_expert_repr, doc_expert_repr, qid, qw, didt, dwt,
      query_cls_repr, doc_cls_repr)
    return out.reshape(B)
